# baseline (speedup 1.0000x reference)
import jax
import jax.numpy as jnp
import numpy as np
from jax import lax
from jax.experimental import pallas as pl
from jax.experimental.pallas import tpu as pltpu
from jax.experimental.pallas import tpu_sc as plsc

_D = 64
_LAB = 1000
_B, _S = 4096, 200
_N = _B * _S
_NC, _NS = 2, 16
_NW = _NC * _NS
_NPW = _N // _NW
_C = 64
_NCHUNK = _NPW // _C
_G16 = _C // 16

def _shuf(ncols):
    p = []
    for blk in range(ncols // 32):
        for i in range(16):
            p.extend((blk * 32 + i, blk * 32 + 16 + i))
    return np.asarray(p, np.int32)

_PERM128 = _shuf(128)
_PERM192 = _shuf(192)


def _lane_bcast(v, idx):
    return lax.gather(
        v, idx[:, None],
        lax.GatherDimensionNumbers(offset_dims=(), collapsed_slice_dims=(0,),
                                   start_index_map=(0,)),
        slice_sizes=(1,), mode=lax.GatherScatterMode.PROMISE_IN_BOUNDS)


def _gram_body(lab_ref, out_ref):
    lab = lab_ref[...]
    out_ref[...] = lax.dot_general(
        lab, lab, (((1,), (1,)), ((), ())), preferred_element_type=jnp.float32)


def _unpack2(v):
    return plsc.unpack(v, format=plsc.PackFormat.INTERLEAVED)


def _asbf(v):
    return plsc.bitcast(v, jnp.bfloat16)


def _sc_body(e2_hbm, rp_hbm, gf_hbm, idx_hbm, out_hbm,
             idx0, idx1, idx2, idx3,
             g1b0, g1b1, g1b2, g1b3, g2b0, g2b1, g2b2, g2b3,
             e2s0, e2s1, e2s2, e2s3, e2o0, e2o1, e2o2, e2o3,
             rpr0, rpr1, rpr2, rpr3,
             g1v0, g1v1, g1v2, g1v3, g2v0, g2v1, g2v2, g2v3,
             out0, out1, out2, out3,
             sem_i, sem_g, sem_o):
    wid = lax.axis_index("s") * _NC + lax.axis_index("c")
    cbase = wid * _NCHUNK

    idxb = (idx0, idx1, idx2, idx3)
    g1b = (g1b0, g1b1, g1b2, g1b3)
    g2b = (g2b0, g2b1, g2b2, g2b3)
    e2sb = (e2s0, e2s1, e2s2, e2s3)
    e2ob = (e2o0, e2o1, e2o2, e2o3)
    rprb = (rpr0, rpr1, rpr2, rpr3)
    g1vb = (g1v0, g1v1, g1v2, g1v3)
    g2vb = (g2v0, g2v1, g2v2, g2v3)
    outb = (out0, out1, out2, out3)

    def fetch_idx(cur, p):
        pltpu.async_copy(idx_hbm.at[cbase + cur], idxb[p], sem_i).wait()
        for g in range(_G16):
            sl = pl.ds(g * 16, 16)
            g1b[p][sl] = idxb[p][3, sl] * _LAB + idxb[p][4, sl]
            g2b[p][sl] = idxb[p][5, sl] * _LAB + idxb[p][6, sl]

    def fire_gathers(p):
        pltpu.async_copy(e2_hbm.at[idxb[p].at[0]], e2sb[p], sem_g)
        pltpu.async_copy(e2_hbm.at[idxb[p].at[1]], e2ob[p], sem_g)
        pltpu.async_copy(rp_hbm.at[idxb[p].at[2]], rprb[p], sem_g)
        pltpu.async_copy(gf_hbm.at[g1b[p]], g1vb[p], sem_g)
        pltpu.async_copy(gf_hbm.at[g2b[p]], g2vb[p], sem_g)

    def wait_gathers(p):
        pltpu.make_async_copy(e2_hbm.at[idxb[p].at[0]], e2sb[p], sem_g).wait()
        pltpu.make_async_copy(e2_hbm.at[idxb[p].at[1]], e2ob[p], sem_g).wait()
        pltpu.make_async_copy(rp_hbm.at[idxb[p].at[2]], rprb[p], sem_g).wait()
        pltpu.make_async_copy(gf_hbm.at[g1b[p]], g1vb[p], sem_g).wait()
        pltpu.make_async_copy(gf_hbm.at[g2b[p]], g2vb[p], sem_g).wait()

    lanes = lax.iota(jnp.int32, 16)
    top = jnp.full((16,), 15, jnp.int32)

    def compute(cur, p):
        def group(g, carry2):
            colb = jnp.zeros((16,), jnp.float32)
            colh = jnp.zeros((16,), jnp.float32)
            colt = jnp.zeros((16,), jnp.float32)
            for e16 in range(16):
                e = g * 16 + e16
                es0 = e2sb[p][e, pl.ds(0, 16)]
                es1 = e2sb[p][e, pl.ds(16, 16)]
                es2 = e2sb[p][e, pl.ds(32, 16)]
                es3 = e2sb[p][e, pl.ds(48, 16)]
                ets0 = e2sb[p][e, pl.ds(64, 16)]
                ets1 = e2sb[p][e, pl.ds(80, 16)]
                ets2 = e2sb[p][e, pl.ds(96, 16)]
                ets3 = e2sb[p][e, pl.ds(112, 16)]
                eo0 = e2ob[p][e, pl.ds(0, 16)]
                eo1 = e2ob[p][e, pl.ds(16, 16)]
                eo2 = e2ob[p][e, pl.ds(32, 16)]
                eo3 = e2ob[p][e, pl.ds(48, 16)]
                eto0 = e2ob[p][e, pl.ds(64, 16)]
                eto1 = e2ob[p][e, pl.ds(80, 16)]
                eto2 = e2ob[p][e, pl.ds(96, 16)]
                eto3 = e2ob[p][e, pl.ds(112, 16)]
                rb0, rb1 = _unpack2(_asbf(rprb[p][e, pl.ds(0, 16)]))
                rb2, rb3 = _unpack2(_asbf(rprb[p][e, pl.ds(16, 16)]))
                rh0, rh1 = _unpack2(_asbf(rprb[p][e, pl.ds(32, 16)]))
                rh2, rh3 = _unpack2(_asbf(rprb[p][e, pl.ds(48, 16)]))
                rt0, rt1 = _unpack2(_asbf(rprb[p][e, pl.ds(64, 16)]))
                rt2, rt3 = _unpack2(_asbf(rprb[p][e, pl.ds(80, 16)]))
                b = es0 * rb0 * eo0 + es1 * rb1 * eo1
                b = b + es2 * rb2 * eo2 + es3 * rb3 * eo3
                h = ets0 * rh0 + ets1 * rh1 + ets2 * rh2 + ets3 * rh3
                t = eto0 * rt0 + eto1 * rt1 + eto2 * rt2 + eto3 * rt3
                mask = lanes == e16
                colb = jnp.where(mask, _lane_bcast(plsc.cumsum(b), top), colb)
                colh = jnp.where(mask, _lane_bcast(plsc.cumsum(h), top), colh)
                colt = jnp.where(mask, _lane_bcast(plsc.cumsum(t), top), colt)
            sl = pl.ds(g * 16, 16)
            ah = colh + g1vb[p][sl]
            at = colt + g2vb[p][sl]
            pb = 1.0 / (1.0 + jnp.exp(-colb))
            ph = 1.0 / (1.0 + jnp.exp(-ah))
            pt = 1.0 / (1.0 + jnp.exp(-at))
            outb[p][sl] = pb * ph * pt
            return carry2

        lax.fori_loop(0, _G16, group, 0)
        base = wid * _NPW + cur * _C
        pltpu.async_copy(outb[p], out_hbm.at[pl.ds(base, _C)], sem_o)

    for j in range(3):
        fetch_idx(j, j)
        fire_gathers(j)

    def quad(i, carry):
        for p in range(4):
            cur = i * 4 + p
            wait_gathers(p)
            nxt = cur + 3
            q = (p + 3) % 4

            @pl.when(nxt < _NCHUNK)
            def _():
                fetch_idx(nxt, q)
                fire_gathers(q)

            @pl.when(cur >= 4)
            def _():
                pltpu.make_async_copy(outb[p], out_hbm.at[pl.ds(0, _C)],
                                      sem_o).wait()

            compute(cur, p)
        return carry

    lax.fori_loop(0, _NCHUNK // 4, quad, 0)
    for j in range(4):
        pltpu.make_async_copy(outb[j], out_hbm.at[pl.ds(0, _C)], sem_o).wait()


def kernel(s, r, o, r_d, r_r, t_s, t_o, E, R, E_t, label_t, R_ht, R_tt):
    gram = pl.pallas_call(
        _gram_body,
        out_shape=jax.ShapeDtypeStruct((_LAB, _LAB), jnp.float32),
    )(label_t)
    gf = gram.reshape(_LAB * _LAB)

    e2 = jnp.concatenate([E, E_t], axis=1)
    rp = jnp.concatenate([R, R_ht, R_tt], axis=1)[:, _PERM192]
    rp = lax.bitcast_convert_type(
        rp.astype(jnp.bfloat16).reshape(_LAB, 96, 2), jnp.int32)
    rp = jnp.concatenate([rp, jnp.zeros((_LAB, 32), jnp.int32)], axis=1)

    idx7 = jnp.stack([s.reshape(_N), o.reshape(_N), r.reshape(_N),
                      t_s.reshape(_N), r_d.reshape(_N),
                      t_o.reshape(_N), r_r.reshape(_N)])
    idx7 = idx7.reshape(7, _N // _C, _C).transpose(1, 0, 2)

    mesh = plsc.VectorSubcoreMesh(core_axis_name="c", subcore_axis_name="s")
    sck = pl.kernel(
        _sc_body,
        out_type=jax.ShapeDtypeStruct((_N,), jnp.float32),
        mesh=mesh,
        compiler_params=pltpu.CompilerParams(needs_layout_passes=False),
        scratch_types=(
            [pltpu.VMEM((7, _C), jnp.int32)] * 4
            + [pltpu.VMEM((_C,), jnp.int32)] * 8
            + [pltpu.VMEM((_C, 128), jnp.float32)] * 8
            + [pltpu.VMEM((_C, 128), jnp.int32)] * 4
            + [pltpu.VMEM((_C,), jnp.float32)] * 8
            + [pltpu.VMEM((_C,), jnp.float32)] * 4
            + [pltpu.SemaphoreType.DMA] * 3
        ),
    )
    out = sck(e2, rp, gf, idx7)
    return out.reshape(_B, _S)


# speedup vs baseline: 1.4469x; 1.4469x over previous
import jax
import jax.numpy as jnp
import numpy as np
from jax import lax
from jax.experimental import pallas as pl
from jax.experimental.pallas import tpu as pltpu
from jax.experimental.pallas import tpu_sc as plsc

_D = 64
_LAB = 1000
_B, _S = 4096, 200
_N = _B * _S
_NC, _NS = 2, 16
_NW = _NC * _NS
_NPW = _N // _NW
_C = 128
_NCHUNK = _NPW // _C
_G16 = _C // 16

def _shuf(ncols):
    p = []
    for blk in range(ncols // 32):
        for i in range(16):
            p.extend((blk * 32 + i, blk * 32 + 16 + i))
    return np.asarray(p, np.int32)

_PERM128 = _shuf(128)
_PERM192 = _shuf(192)


def _lane_bcast(v, idx):
    return lax.gather(
        v, idx[:, None],
        lax.GatherDimensionNumbers(offset_dims=(), collapsed_slice_dims=(0,),
                                   start_index_map=(0,)),
        slice_sizes=(1,), mode=lax.GatherScatterMode.PROMISE_IN_BOUNDS)


def _gram_body(lab_ref, out_ref):
    lab = lab_ref[...]
    out_ref[...] = lax.dot_general(
        lab, lab, (((1,), (1,)), ((), ())), preferred_element_type=jnp.float32)


def _unpack2(v):
    return plsc.unpack(v, format=plsc.PackFormat.INTERLEAVED)


def _asbf(v):
    return plsc.bitcast(v, jnp.bfloat16)


def _sc_body(e2_hbm, rp_hbm, gf_hbm, idx_hbm, out_hbm,
             idx0, idx1, g1b0, g1b1, g2b0, g2b1,
             e2s0, e2s1, e2o0, e2o1, rpr0, rpr1,
             g1v0, g1v1, g2v0, g2v1, out0, out1,
             sem_i, sem_g, sem_o):
    wid = lax.axis_index("s") * _NC + lax.axis_index("c")
    cbase = wid * _NCHUNK

    idxb = (idx0, idx1)
    g1b = (g1b0, g1b1)
    g2b = (g2b0, g2b1)
    e2sb = (e2s0, e2s1)
    e2ob = (e2o0, e2o1)
    rprb = (rpr0, rpr1)
    g1vb = (g1v0, g1v1)
    g2vb = (g2v0, g2v1)
    outb = (out0, out1)

    def fire_idx(cur, p):
        pltpu.async_copy(idx_hbm.at[cbase + cur], idxb[p], sem_i)

    def wait_fuse_idx(p):
        pltpu.make_async_copy(idx_hbm.at[cbase], idxb[p], sem_i).wait()
        for g in range(_G16):
            sl = pl.ds(g * 16, 16)
            g1b[p][sl] = idxb[p][3, sl] * _LAB + idxb[p][4, sl]
            g2b[p][sl] = idxb[p][5, sl] * _LAB + idxb[p][6, sl]

    def fire_gathers(p):
        pltpu.async_copy(e2_hbm.at[idxb[p].at[0]], e2sb[p], sem_g)
        pltpu.async_copy(e2_hbm.at[idxb[p].at[1]], e2ob[p], sem_g)
        pltpu.async_copy(rp_hbm.at[idxb[p].at[2]], rprb[p], sem_g)
        pltpu.async_copy(gf_hbm.at[g1b[p]], g1vb[p], sem_g)
        pltpu.async_copy(gf_hbm.at[g2b[p]], g2vb[p], sem_g)

    def wait_gathers(p):
        pltpu.make_async_copy(e2_hbm.at[idxb[p].at[0]], e2sb[p], sem_g).wait()
        pltpu.make_async_copy(e2_hbm.at[idxb[p].at[1]], e2ob[p], sem_g).wait()
        pltpu.make_async_copy(rp_hbm.at[idxb[p].at[2]], rprb[p], sem_g).wait()
        pltpu.make_async_copy(gf_hbm.at[g1b[p]], g1vb[p], sem_g).wait()
        pltpu.make_async_copy(gf_hbm.at[g2b[p]], g2vb[p], sem_g).wait()

    lanes = lax.iota(jnp.int32, 16)
    top = jnp.full((16,), 15, jnp.int32)

    def compute(cur, p):
        def group(g, carry2):
            colb = jnp.zeros((16,), jnp.float32)
            colh = jnp.zeros((16,), jnp.float32)
            colt = jnp.zeros((16,), jnp.float32)
            for e16 in range(16):
                e = g * 16 + e16
                es0 = e2sb[p][e, pl.ds(0, 16)]
                es1 = e2sb[p][e, pl.ds(16, 16)]
                es2 = e2sb[p][e, pl.ds(32, 16)]
                es3 = e2sb[p][e, pl.ds(48, 16)]
                ets0 = e2sb[p][e, pl.ds(64, 16)]
                ets1 = e2sb[p][e, pl.ds(80, 16)]
                ets2 = e2sb[p][e, pl.ds(96, 16)]
                ets3 = e2sb[p][e, pl.ds(112, 16)]
                eo0 = e2ob[p][e, pl.ds(0, 16)]
                eo1 = e2ob[p][e, pl.ds(16, 16)]
                eo2 = e2ob[p][e, pl.ds(32, 16)]
                eo3 = e2ob[p][e, pl.ds(48, 16)]
                eto0 = e2ob[p][e, pl.ds(64, 16)]
                eto1 = e2ob[p][e, pl.ds(80, 16)]
                eto2 = e2ob[p][e, pl.ds(96, 16)]
                eto3 = e2ob[p][e, pl.ds(112, 16)]
                rb0, rb1 = _unpack2(_asbf(rprb[p][e, pl.ds(0, 16)]))
                rb2, rb3 = _unpack2(_asbf(rprb[p][e, pl.ds(16, 16)]))
                rh0, rh1 = _unpack2(_asbf(rprb[p][e, pl.ds(32, 16)]))
                rh2, rh3 = _unpack2(_asbf(rprb[p][e, pl.ds(48, 16)]))
                rt0, rt1 = _unpack2(_asbf(rprb[p][e, pl.ds(64, 16)]))
                rt2, rt3 = _unpack2(_asbf(rprb[p][e, pl.ds(80, 16)]))
                b = es0 * rb0 * eo0 + es1 * rb1 * eo1
                b = b + es2 * rb2 * eo2 + es3 * rb3 * eo3
                h = ets0 * rh0 + ets1 * rh1 + ets2 * rh2 + ets3 * rh3
                t = eto0 * rt0 + eto1 * rt1 + eto2 * rt2 + eto3 * rt3
                mask = lanes == e16
                colb = jnp.where(mask, _lane_bcast(plsc.cumsum(b), top), colb)
                colh = jnp.where(mask, _lane_bcast(plsc.cumsum(h), top), colh)
                colt = jnp.where(mask, _lane_bcast(plsc.cumsum(t), top), colt)
            sl = pl.ds(g * 16, 16)
            ah = colh + g1vb[p][sl]
            at = colt + g2vb[p][sl]
            pb = 1.0 / (1.0 + jnp.exp(-colb))
            ph = 1.0 / (1.0 + jnp.exp(-ah))
            pt = 1.0 / (1.0 + jnp.exp(-at))
            outb[p][sl] = pb * ph * pt
            return carry2

        lax.fori_loop(0, _G16, group, 0)
        base = wid * _NPW + cur * _C
        pltpu.async_copy(outb[p], out_hbm.at[pl.ds(base, _C)], sem_o)

    fire_idx(0, 0)
    wait_fuse_idx(0)
    fire_gathers(0)
    fire_idx(1, 1)

    def pair(i, carry):
        for p in (0, 1):
            cur = i * 2 + p
            q = 1 - p

            @pl.when(cur + 1 < _NCHUNK)
            def _():
                wait_fuse_idx(q)
                fire_gathers(q)

            wait_gathers(p)

            @pl.when(cur + 2 < _NCHUNK)
            def _():
                fire_idx(cur + 2, p)

            @pl.when(cur >= 2)
            def _():
                pltpu.make_async_copy(outb[p], out_hbm.at[pl.ds(0, _C)],
                                      sem_o).wait()

            compute(cur, p)
        return carry

    lax.fori_loop(0, _NCHUNK // 2, pair, 0)
    pltpu.make_async_copy(out0, out_hbm.at[pl.ds(0, _C)], sem_o).wait()
    pltpu.make_async_copy(out1, out_hbm.at[pl.ds(0, _C)], sem_o).wait()


def kernel(s, r, o, r_d, r_r, t_s, t_o, E, R, E_t, label_t, R_ht, R_tt):
    gram = pl.pallas_call(
        _gram_body,
        out_shape=jax.ShapeDtypeStruct((_LAB, _LAB), jnp.float32),
    )(label_t)
    gf = gram.reshape(_LAB * _LAB)

    e2 = jnp.concatenate([E, E_t], axis=1)
    rp = jnp.concatenate([R, R_ht, R_tt], axis=1)[:, _PERM192]
    rp = lax.bitcast_convert_type(
        rp.astype(jnp.bfloat16).reshape(_LAB, 96, 2), jnp.int32)
    rp = jnp.concatenate([rp, jnp.zeros((_LAB, 32), jnp.int32)], axis=1)

    idx7 = jnp.stack([s.reshape(_N), o.reshape(_N), r.reshape(_N),
                      t_s.reshape(_N), r_d.reshape(_N),
                      t_o.reshape(_N), r_r.reshape(_N)])
    idx7 = idx7.reshape(7, _N // _C, _C).transpose(1, 0, 2)

    mesh = plsc.VectorSubcoreMesh(core_axis_name="c", subcore_axis_name="s")
    sck = pl.kernel(
        _sc_body,
        out_type=jax.ShapeDtypeStruct((_N,), jnp.float32),
        mesh=mesh,
        compiler_params=pltpu.CompilerParams(needs_layout_passes=False),
        scratch_types=[
            pltpu.VMEM((7, _C), jnp.int32),
            pltpu.VMEM((7, _C), jnp.int32),
            pltpu.VMEM((_C,), jnp.int32),
            pltpu.VMEM((_C,), jnp.int32),
            pltpu.VMEM((_C,), jnp.int32),
            pltpu.VMEM((_C,), jnp.int32),
            pltpu.VMEM((_C, 128), jnp.float32),
            pltpu.VMEM((_C, 128), jnp.float32),
            pltpu.VMEM((_C, 128), jnp.float32),
            pltpu.VMEM((_C, 128), jnp.float32),
            pltpu.VMEM((_C, 128), jnp.int32),
            pltpu.VMEM((_C, 128), jnp.int32),
            pltpu.VMEM((_C,), jnp.float32),
            pltpu.VMEM((_C,), jnp.float32),
            pltpu.VMEM((_C,), jnp.float32),
            pltpu.VMEM((_C,), jnp.float32),
            pltpu.VMEM((_C,), jnp.float32),
            pltpu.VMEM((_C,), jnp.float32),
            pltpu.SemaphoreType.DMA,
            pltpu.SemaphoreType.DMA,
            pltpu.SemaphoreType.DMA,
        ],
    )
    out = sck(e2, rp, gf, idx7)
    return out.reshape(_B, _S)
